# baseline (speedup 1.0000x reference)
import functools

import jax
import jax.numpy as jnp
from jax import lax
from jax.experimental import pallas as pl
from jax.experimental.pallas import tpu as pltpu
from jax.experimental.pallas import tpu_sc as plsc

IN_FEATURES = 1000000
OUT_FEATURES = 64
BATCH = 16384
NUM_FIELDS = 26

_INFO = plsc.get_sparse_core_info()
NC = _INFO.num_cores
NS = _INFO.num_subcores
NW = NC * NS
BPW = BATCH // NW
SB = 128
NSB = BPW // SB
LANES = 16
CPD = OUT_FEATURES // LANES


def _body(idx_hbm, w_hbm, bias_hbm, out_hbm, idx_v, rows_v, acc_v, bias_v, sem):
    wid = lax.axis_index("s") * NC + lax.axis_index("c")
    base = wid * BPW

    pltpu.sync_copy(bias_hbm, bias_v)
    pltpu.sync_copy(idx_hbm.at[:, pl.ds(base, BPW)], idx_v)

    def do_subblock(s, carry):
        row0 = s * SB

        pltpu.async_copy(
            w_hbm.at[idx_v.at[0, pl.ds(row0, SB)]], rows_v, sem
        ).wait()

        def init_row(r, c2):
            for c in range(CPD):
                sl = pl.ds(c * LANES, LANES)
                acc_v[r, sl] = rows_v[r, sl] + bias_v[sl]
            return c2

        lax.fori_loop(0, SB, init_row, 0)

        for f in range(1, NUM_FIELDS):
            pltpu.async_copy(
                w_hbm.at[idx_v.at[f, pl.ds(row0, SB)]], rows_v, sem
            ).wait()

            def acc_row(r, c2):
                for c in range(CPD):
                    sl = pl.ds(c * LANES, LANES)
                    acc_v[r, sl] = acc_v[r, sl] + rows_v[r, sl]
                return c2

            lax.fori_loop(0, SB, acc_row, 0)

        pltpu.sync_copy(acc_v, out_hbm.at[pl.ds(base + row0, SB)])
        return carry

    lax.fori_loop(0, NSB, do_subblock, 0)


@jax.jit
def _run(idx_t, weight, bias):
    kern = pl.kernel(
        _body,
        mesh=plsc.VectorSubcoreMesh(core_axis_name="c", subcore_axis_name="s"),
        compiler_params=pltpu.CompilerParams(use_tc_tiling_on_sc=False),
        out_type=jax.ShapeDtypeStruct((BATCH, OUT_FEATURES), jnp.float32),
        scratch_types=[
            pltpu.VMEM((NUM_FIELDS, BPW), jnp.int32),
            pltpu.VMEM((SB, OUT_FEATURES), jnp.float32),
            pltpu.VMEM((SB, OUT_FEATURES), jnp.float32),
            pltpu.VMEM((OUT_FEATURES,), jnp.float32),
            pltpu.SemaphoreType.DMA,
        ],
    )
    return kern(idx_t, weight, bias)


def kernel(indices, weight, bias):
    idx_t = jnp.asarray(indices, dtype=jnp.int32).T
    return _run(idx_t, weight, bias)


# speedup vs baseline: 1.1431x; 1.1431x over previous
import jax
import jax.numpy as jnp
from jax import lax
from jax.experimental import pallas as pl
from jax.experimental.pallas import tpu as pltpu
from jax.experimental.pallas import tpu_sc as plsc

IN_FEATURES = 1000000
OUT_FEATURES = 64
BATCH = 16384
NUM_FIELDS = 26

_INFO = plsc.get_sparse_core_info()
NC = _INFO.num_cores
NS = _INFO.num_subcores
NW = NC * NS
BPW = BATCH // NW
SB = 128
NSB = BPW // SB
LANES = 16
CPD = OUT_FEATURES // LANES
K = 4


def _body(idx_hbm, w_hbm, bias_hbm, out_hbm, idx_v, rows_v, acc_v, bias_v,
          sem0, sem1, sem2, sem3, out_sem):
    sems = (sem0, sem1, sem2, sem3)
    wid = lax.axis_index("s") * NC + lax.axis_index("c")
    base = wid * BPW

    pltpu.sync_copy(bias_hbm, bias_v)
    pltpu.sync_copy(idx_hbm.at[:, pl.ds(base, BPW)], idx_v)

    bias_regs = [bias_v[pl.ds(c * LANES, LANES)] for c in range(CPD)]

    tasks = [(s, f) for s in range(NSB) for f in range(NUM_FIELDS)]

    def fire(g):
        s, f = tasks[g]
        slot = g % K
        return pltpu.async_copy(
            w_hbm.at[idx_v.at[f, pl.ds(s * SB, SB)]],
            rows_v.at[slot],
            sems[slot],
        )

    handles = {}
    for g in range(K):
        handles[g] = fire(g)

    out_copies = []
    for g in range(len(tasks)):
        s, f = tasks[g]
        slot = g % K
        handles[g].wait()
        if g + K < len(tasks):
            handles[g + K] = fire(g + K)

        row0 = s * SB
        if f == 0:
            @plsc.parallel_loop(0, SB, unroll=4)
            def _init(r):
                for c in range(CPD):
                    sl = pl.ds(c * LANES, LANES)
                    acc_v[row0 + r, sl] = rows_v[slot, r, sl] + bias_regs[c]
        else:
            @plsc.parallel_loop(0, SB, unroll=4)
            def _acc(r):
                for c in range(CPD):
                    sl = pl.ds(c * LANES, LANES)
                    plsc.addupdate(acc_v.at[row0 + r, sl], rows_v[slot, r, sl])

        if f == NUM_FIELDS - 1:
            out_copies.append(
                pltpu.async_copy(
                    acc_v.at[pl.ds(row0, SB)],
                    out_hbm.at[pl.ds(base + row0, SB)],
                    out_sem,
                )
            )

    for cp in out_copies:
        cp.wait()


@jax.jit
def _run(idx_t, weight, bias):
    kern = pl.kernel(
        _body,
        mesh=plsc.VectorSubcoreMesh(core_axis_name="c", subcore_axis_name="s"),
        compiler_params=pltpu.CompilerParams(use_tc_tiling_on_sc=False),
        out_type=jax.ShapeDtypeStruct((BATCH, OUT_FEATURES), jnp.float32),
        scratch_types=[
            pltpu.VMEM((NUM_FIELDS, BPW), jnp.int32),
            pltpu.VMEM((K, SB, OUT_FEATURES), jnp.float32),
            pltpu.VMEM((BPW, OUT_FEATURES), jnp.float32),
            pltpu.VMEM((OUT_FEATURES,), jnp.float32),
            pltpu.SemaphoreType.DMA,
            pltpu.SemaphoreType.DMA,
            pltpu.SemaphoreType.DMA,
            pltpu.SemaphoreType.DMA,
            pltpu.SemaphoreType.DMA,
        ],
    )
    return kern(idx_t, weight, bias)


def kernel(indices, weight, bias):
    idx_t = jnp.asarray(indices, dtype=jnp.int32).T
    return _run(idx_t, weight, bias)
